# baseline (speedup 1.0000x reference)
import functools

import jax
import jax.numpy as jnp
from jax.experimental import pallas as pl
from jax.experimental.pallas import tpu as pltpu

K = 32
B = 64
N = 32768
D = 64
BN = 4096
NB = N // BN


def _fused_kernel(x_ref, a_ref, bmat_ref, o_ref, s_ref):
    b = pl.program_id(0)
    nb = pl.program_id(1)

    x = x_ref[0]
    proj = jnp.dot(x, bmat_ref[...], preferred_element_type=jnp.float32)
    y = proj * x
    ones = jnp.ones((1, D), jnp.float32)
    pred = jax.lax.dot_general(ones, y, (((1,), (1,)), ((), ())),
                               preferred_element_type=jnp.float32)
    score = a_ref[0] * pred
    s_ref[pl.ds(b, 1), pl.ds(nb * BN, BN)] = score

    @pl.when(jnp.logical_and(b == B - 1, nb == NB - 1))
    def _topk():
        def step(i, carry):
            total, consumed = carry
            s = s_ref[...]
            m = jnp.max(s, axis=1, keepdims=True)
            eq = (s == m)
            cnt = jnp.sum(eq.astype(jnp.float32), axis=1, keepdims=True)
            take = jnp.clip(jnp.float32(K) - consumed, 0.0, cnt)
            total = total + jnp.where(take > 0.0, m, 0.0) * take
            consumed = consumed + take
            s_ref[...] = jnp.where(eq, -jnp.inf, s)
            return total, consumed

        z = jnp.zeros((B, 1), jnp.float32)
        total, _ = jax.lax.fori_loop(0, K, step, (z, z))
        o_ref[...] = total * (1.0 / K)


@functools.partial(jax.jit, static_argnames=())
def kernel(ins_emb, attn, W):
    d = W.shape[0]
    bmat = jnp.zeros((D, D), jnp.float32).at[:d, d:].set(W.T)
    attn2 = attn.reshape(B * NB, 1, BN)

    out = pl.pallas_call(
        _fused_kernel,
        grid=(B, NB),
        in_specs=[
            pl.BlockSpec((1, BN, D), lambda b, nb: (b, nb, 0)),
            pl.BlockSpec((1, 1, BN), lambda b, nb: (b * NB + nb, 0, 0)),
            pl.BlockSpec((D, D), lambda b, nb: (0, 0)),
        ],
        out_specs=pl.BlockSpec((B, 1), lambda b, nb: (0, 0)),
        out_shape=jax.ShapeDtypeStruct((B, 1), jnp.float32),
        scratch_shapes=[pltpu.VMEM((B, N), jnp.float32)],
    )(ins_emb, attn2, bmat)
    return out
